# Initial kernel scaffold; baseline (speedup 1.0000x reference)
import jax
import jax.numpy as jnp
from jax.experimental import pallas as pl


def kernel(X, ref_a, ref_b):
    raise NotImplementedError("write your pallas kernel here")


# speedup vs baseline: 65.8152x; 65.8152x over previous
import functools

import jax
import jax.numpy as jnp
from jax import lax
from jax.experimental import pallas as pl
from jax.experimental.pallas import tpu as pltpu
from jax.experimental.pallas import tpu_sc as plsc

NC = 2
NS = 16
L = 16


def _rsqrt_f32(x):
    xi = plsc.bitcast(x, jnp.int32)
    y = plsc.bitcast(jnp.int32(0x5F3759DF) - (xi >> 1), jnp.float32)
    for _ in range(3):
        y = y * (1.5 - 0.5 * x * y * y)
    return y


@functools.partial(jax.jit, static_argnames=("n_nodes", "n_edges"))
def _norm_sc(ref_a, ref_b, *, n_nodes, n_edges):
    eh = n_edges // NS
    eo = n_edges // (NC * NS)

    mesh = plsc.VectorSubcoreMesh(core_axis_name="c", subcore_axis_name="s")

    @functools.partial(
        pl.kernel,
        out_type=jax.ShapeDtypeStruct((n_edges,), jnp.float32),
        mesh=mesh,
        compiler_params=pltpu.CompilerParams(needs_layout_passes=False),
        scratch_types=[
            pltpu.VMEM_SHARED((n_nodes,), jnp.float32),
            pltpu.VMEM_SHARED((n_nodes,), jnp.float32),
            pltpu.VMEM((eh,), jnp.int32),
            pltpu.VMEM((eh,), jnp.int32),
            pltpu.VMEM((eh,), jnp.float32),
            pltpu.VMEM((eo,), jnp.int32),
            pltpu.VMEM((eo,), jnp.int32),
            pltpu.VMEM((n_nodes,), jnp.float32),
            pltpu.VMEM((n_nodes,), jnp.float32),
            pltpu.VMEM((eo,), jnp.float32),
        ],
    )
    def norm_kernel(a_hbm, b_hbm, out_hbm, deg_a_sh, deg_b_sh,
                    ia_v, ib_v, ones_v, ea_v, eb_v, da_v, db_v, out_v):
        c = lax.axis_index("c")
        s = lax.axis_index("s")
        wid = c * NS + s

        def fill_ones(i, _):
            ones_v[pl.ds(i * L, L)] = jnp.full((L,), 1.0, jnp.float32)
            return 0

        lax.fori_loop(0, eh // L, fill_ones, 0)

        def fill_zero(i, _):
            da_v[pl.ds(i * L, L)] = jnp.zeros((L,), jnp.float32)
            return 0

        lax.fori_loop(0, n_nodes // L, fill_zero, 0)

        @pl.when(s == 0)
        def _():
            pltpu.sync_copy(da_v, deg_a_sh)

        @pl.when(s == 1)
        def _():
            pltpu.sync_copy(da_v, deg_b_sh)

        plsc.subcore_barrier()

        base_h = s * eh
        pltpu.sync_copy(a_hbm.at[pl.ds(base_h, eh)], ia_v)
        pltpu.sync_copy(b_hbm.at[pl.ds(base_h, eh)], ib_v)
        pltpu.sync_copy(ones_v, deg_a_sh.at[ia_v], add=True)
        pltpu.sync_copy(ones_v, deg_b_sh.at[ib_v], add=True)
        plsc.subcore_barrier()

        pltpu.sync_copy(deg_a_sh, da_v)
        pltpu.sync_copy(deg_b_sh, db_v)

        base_o = wid * eo
        pltpu.sync_copy(a_hbm.at[pl.ds(base_o, eo)], ea_v)
        pltpu.sync_copy(b_hbm.at[pl.ds(base_o, eo)], eb_v)

        def edge_body(i, _):
            sl = pl.ds(i * L, L)
            da = plsc.load_gather(da_v, [ea_v[sl]])
            db = plsc.load_gather(db_v, [eb_v[sl]])
            out_v[sl] = _rsqrt_f32(da * db)
            return 0

        lax.fori_loop(0, eo // L, edge_body, 0)

        pltpu.sync_copy(out_v, out_hbm.at[pl.ds(base_o, eo)])

    return norm_kernel(ref_a, ref_b)


def kernel(X, ref_a, ref_b):
    n_nodes = X.shape[0]
    n_edges = ref_a.shape[0]
    return _norm_sc(
        ref_a.astype(jnp.int32),
        ref_b.astype(jnp.int32),
        n_nodes=n_nodes,
        n_edges=n_edges,
    )


# speedup vs baseline: 78.9732x; 1.1999x over previous
import functools

import jax
import jax.numpy as jnp
from jax import lax
from jax.experimental import pallas as pl
from jax.experimental.pallas import tpu as pltpu
from jax.experimental.pallas import tpu_sc as plsc

NC = 2
NS = 16
L = 16


def _rsqrt_f32(x):
    xi = plsc.bitcast(x, jnp.int32)
    y = plsc.bitcast(jnp.int32(0x5F3759DF) - (xi >> 1), jnp.float32)
    for _ in range(2):
        y = y * (1.5 - 0.5 * x * y * y)
    return y


@functools.partial(jax.jit, static_argnames=("n_nodes", "n_edges"))
def _norm_sc(ref_a, ref_b, ones, zeros, *, n_nodes, n_edges):
    eh = n_edges // NS
    eo = n_edges // (NC * NS)

    mesh = plsc.VectorSubcoreMesh(core_axis_name="c", subcore_axis_name="s")

    @functools.partial(
        pl.kernel,
        out_type=jax.ShapeDtypeStruct((n_edges,), jnp.float32),
        mesh=mesh,
        compiler_params=pltpu.CompilerParams(needs_layout_passes=False),
        scratch_types=[
            pltpu.VMEM_SHARED((n_nodes,), jnp.float32),
            pltpu.VMEM_SHARED((n_nodes,), jnp.float32),
            pltpu.VMEM((eh,), jnp.int32),
            pltpu.VMEM((eh,), jnp.int32),
            pltpu.VMEM((eh,), jnp.float32),
            pltpu.VMEM((eo,), jnp.int32),
            pltpu.VMEM((eo,), jnp.int32),
            pltpu.VMEM((n_nodes,), jnp.float32),
            pltpu.VMEM((n_nodes,), jnp.float32),
            pltpu.VMEM((eo,), jnp.float32),
            pltpu.SemaphoreType.DMA,
            pltpu.SemaphoreType.DMA,
            pltpu.SemaphoreType.DMA,
            pltpu.SemaphoreType.DMA,
            pltpu.SemaphoreType.DMA,
            pltpu.SemaphoreType.DMA,
            pltpu.SemaphoreType.DMA,
            pltpu.SemaphoreType.DMA,
            pltpu.SemaphoreType.DMA,
        ],
    )
    def norm_kernel(a_hbm, b_hbm, ones_hbm, zeros_hbm, out_hbm,
                    deg_a_sh, deg_b_sh,
                    ia_v, ib_v, ones_v, ea_v, eb_v, da_v, db_v, out_v,
                    sem_ia, sem_ib, sem_ones, sem_ea, sem_eb,
                    sem_sca, sem_scb, sem_da, sem_db):
        c = lax.axis_index("c")
        s = lax.axis_index("s")
        wid = c * NS + s
        base_h = s * eh
        base_o = wid * eo

        with jax.named_scope("p0_issue_dmas"):
            cp_ia = pltpu.async_copy(a_hbm.at[pl.ds(base_h, eh)], ia_v, sem_ia)
            cp_ib = pltpu.async_copy(b_hbm.at[pl.ds(base_h, eh)], ib_v, sem_ib)
            cp_ones = pltpu.async_copy(ones_hbm, ones_v, sem_ones)
            cp_ea = pltpu.async_copy(a_hbm.at[pl.ds(base_o, eo)], ea_v, sem_ea)
            cp_eb = pltpu.async_copy(b_hbm.at[pl.ds(base_o, eo)], eb_v, sem_eb)

        with jax.named_scope("p1_zero_hist"):
            @pl.when(s == 0)
            def _():
                pltpu.sync_copy(zeros_hbm, deg_a_sh)

            @pl.when(s == 1)
            def _():
                pltpu.sync_copy(zeros_hbm, deg_b_sh)

            plsc.subcore_barrier()

        with jax.named_scope("p2_scatter_add"):
            cp_ia.wait()
            cp_ones.wait()
            cp_sca = pltpu.async_copy(
                ones_v, deg_a_sh.at[ia_v], sem_sca, add=True)
            cp_ib.wait()
            cp_scb = pltpu.async_copy(
                ones_v, deg_b_sh.at[ib_v], sem_scb, add=True)
            cp_sca.wait()
            cp_scb.wait()
            plsc.subcore_barrier()

        with jax.named_scope("p3_snapshot"):
            cp_da = pltpu.async_copy(deg_a_sh, da_v, sem_da)
            cp_db = pltpu.async_copy(deg_b_sh, db_v, sem_db)
            cp_da.wait()
            cp_db.wait()
            cp_ea.wait()
            cp_eb.wait()

        with jax.named_scope("p4_gather_rsqrt"):
            def edge_body(i, _):
                sl = pl.ds(i * L, L)
                da = plsc.load_gather(da_v, [ea_v[sl]])
                db = plsc.load_gather(db_v, [eb_v[sl]])
                out_v[sl] = _rsqrt_f32(da * db)
                return 0

            lax.fori_loop(0, eo // L, edge_body, 0)

        with jax.named_scope("p5_writeback"):
            pltpu.sync_copy(out_v, out_hbm.at[pl.ds(base_o, eo)])

    return norm_kernel(ref_a, ref_b, ones, zeros)


def kernel(X, ref_a, ref_b):
    n_nodes = X.shape[0]
    n_edges = ref_a.shape[0]
    ones = jnp.ones((n_edges // NS,), jnp.float32)
    zeros = jnp.zeros((n_nodes,), jnp.float32)
    return _norm_sc(
        ref_a.astype(jnp.int32),
        ref_b.astype(jnp.int32),
        ones,
        zeros,
        n_nodes=n_nodes,
        n_edges=n_edges,
    )


# speedup vs baseline: 96.6406x; 1.2237x over previous
import functools

import jax
import jax.numpy as jnp
from jax import lax
from jax.experimental import pallas as pl
from jax.experimental.pallas import tpu as pltpu
from jax.experimental.pallas import tpu_sc as plsc

NC = 2
NS = 16
L = 16


def _rsqrt_f32(x):
    xi = plsc.bitcast(x, jnp.int32)
    y = plsc.bitcast(jnp.int32(0x5F3759DF) - (xi >> 1), jnp.float32)
    for _ in range(2):
        y = y * (1.5 - 0.5 * x * y * y)
    return y


@functools.partial(jax.jit, static_argnames=("n_nodes", "n_edges"))
def _norm_sc(ref_a, ref_b, ones, zeros, *, n_nodes, n_edges):
    eh = n_edges // NS
    eo = n_edges // (NC * NS)

    mesh = plsc.VectorSubcoreMesh(core_axis_name="c", subcore_axis_name="s")

    @functools.partial(
        pl.kernel,
        out_type=jax.ShapeDtypeStruct((n_edges,), jnp.float32),
        mesh=mesh,
        compiler_params=pltpu.CompilerParams(needs_layout_passes=False),
        scratch_types=[
            pltpu.VMEM_SHARED((n_nodes,), jnp.float32),
            pltpu.VMEM_SHARED((n_nodes,), jnp.float32),
            pltpu.VMEM((eh,), jnp.int32),
            pltpu.VMEM((eh,), jnp.int32),
            pltpu.VMEM((eh,), jnp.float32),
            pltpu.VMEM((eo,), jnp.int32),
            pltpu.VMEM((eo,), jnp.int32),
            pltpu.VMEM((n_nodes,), jnp.float32),
            pltpu.VMEM((n_nodes,), jnp.float32),
            pltpu.VMEM((eo,), jnp.float32),
            pltpu.SemaphoreType.DMA,
            pltpu.SemaphoreType.DMA,
            pltpu.SemaphoreType.DMA,
            pltpu.SemaphoreType.DMA,
            pltpu.SemaphoreType.DMA,
            pltpu.SemaphoreType.DMA,
            pltpu.SemaphoreType.DMA,
            pltpu.SemaphoreType.DMA,
            pltpu.SemaphoreType.DMA,
        ],
    )
    def norm_kernel(a_hbm, b_hbm, ones_hbm, zeros_hbm, out_hbm,
                    deg_a_sh, deg_b_sh,
                    ia_v, ib_v, ones_v, ea_v, eb_v, da_v, db_v, out_v,
                    sem_ia, sem_ib, sem_ones, sem_ea, sem_eb,
                    sem_sca, sem_scb, sem_da, sem_db):
        c = lax.axis_index("c")
        s = lax.axis_index("s")
        wid = c * NS + s
        base_h = s * eh
        base_o = wid * eo

        cp_ia = pltpu.async_copy(a_hbm.at[pl.ds(base_h, eh)], ia_v, sem_ia)
        cp_ib = pltpu.async_copy(b_hbm.at[pl.ds(base_h, eh)], ib_v, sem_ib)
        cp_ones = pltpu.async_copy(ones_hbm, ones_v, sem_ones)
        cp_ea = pltpu.async_copy(a_hbm.at[pl.ds(base_o, eo)], ea_v, sem_ea)
        cp_eb = pltpu.async_copy(b_hbm.at[pl.ds(base_o, eo)], eb_v, sem_eb)

        @pl.when(s == 0)
        def _():
            pltpu.sync_copy(zeros_hbm, deg_a_sh)

        @pl.when(s == 1)
        def _():
            pltpu.sync_copy(zeros_hbm, deg_b_sh)

        plsc.subcore_barrier()

        cp_ia.wait()
        cp_ones.wait()
        cp_sca = pltpu.async_copy(ones_v, deg_a_sh.at[ia_v], sem_sca, add=True)
        cp_ib.wait()
        cp_scb = pltpu.async_copy(ones_v, deg_b_sh.at[ib_v], sem_scb, add=True)
        cp_sca.wait()
        cp_scb.wait()
        plsc.subcore_barrier()

        cp_da = pltpu.async_copy(deg_a_sh, da_v, sem_da)
        cp_db = pltpu.async_copy(deg_b_sh, db_v, sem_db)
        cp_da.wait()
        cp_db.wait()
        cp_ea.wait()
        cp_eb.wait()

        @plsc.parallel_loop(0, eo, step=L, unroll=8)
        def _(i):
            sl = pl.ds(i, L)
            da = plsc.load_gather(da_v, [ea_v[sl]])
            db = plsc.load_gather(db_v, [eb_v[sl]])
            out_v[sl] = _rsqrt_f32(da * db)

        pltpu.sync_copy(out_v, out_hbm.at[pl.ds(base_o, eo)])

    return norm_kernel(ref_a, ref_b, ones, zeros)


def kernel(X, ref_a, ref_b):
    n_nodes = X.shape[0]
    n_edges = ref_a.shape[0]
    ones = jnp.ones((n_edges // NS,), jnp.float32)
    zeros = jnp.zeros((n_nodes,), jnp.float32)
    return _norm_sc(
        ref_a.astype(jnp.int32),
        ref_b.astype(jnp.int32),
        ones,
        zeros,
        n_nodes=n_nodes,
        n_edges=n_edges,
    )


# speedup vs baseline: 100.1631x; 1.0364x over previous
import functools

import jax
import jax.numpy as jnp
from jax import lax
from jax.experimental import pallas as pl
from jax.experimental.pallas import tpu as pltpu
from jax.experimental.pallas import tpu_sc as plsc

NC = 2
NS = 16
L = 16


def _rsqrt_f32(x):
    xi = plsc.bitcast(x, jnp.int32)
    y = plsc.bitcast(jnp.int32(0x5F3759DF) - (xi >> 1), jnp.float32)
    for _ in range(2):
        y = y * (1.5 - 0.5 * x * y * y)
    return y


@functools.partial(jax.jit, static_argnames=("n_nodes", "n_edges"))
def _norm_sc(ref_a, ref_b, ones, zeros, *, n_nodes, n_edges):
    eo = n_edges // (NC * NS)

    mesh = plsc.VectorSubcoreMesh(core_axis_name="c", subcore_axis_name="s")

    @functools.partial(
        pl.kernel,
        out_type=jax.ShapeDtypeStruct((n_edges,), jnp.float32),
        mesh=mesh,
        compiler_params=pltpu.CompilerParams(needs_layout_passes=False),
        scratch_types=[
            pltpu.VMEM_SHARED((n_nodes,), jnp.float32),
            pltpu.VMEM_SHARED((n_nodes,), jnp.float32),
            pltpu.VMEM((eo,), jnp.int32),
            pltpu.VMEM((eo,), jnp.int32),
            pltpu.VMEM((eo,), jnp.int32),
            pltpu.VMEM((eo,), jnp.int32),
            pltpu.VMEM((eo,), jnp.float32),
            pltpu.VMEM((n_nodes,), jnp.float32),
            pltpu.VMEM((n_nodes,), jnp.float32),
            pltpu.VMEM((eo,), jnp.float32),
            pltpu.SemaphoreType.DMA,
            pltpu.SemaphoreType.DMA,
            pltpu.SemaphoreType.DMA,
            pltpu.SemaphoreType.DMA,
            pltpu.SemaphoreType.DMA,
            pltpu.SemaphoreType.DMA,
            pltpu.SemaphoreType.DMA,
            pltpu.SemaphoreType.DMA,
        ],
    )
    def norm_kernel(a_hbm, b_hbm, ones_hbm, zeros_hbm, out_hbm,
                    deg_a_sh, deg_b_sh,
                    ia1_v, ia2_v, ib1_v, ib2_v, ones_v, da_v, db_v, out_v,
                    sem_ia1, sem_ia2, sem_ib1, sem_ib2, sem_ones,
                    sem_sc, sem_da, sem_db):
        c = lax.axis_index("c")
        s = lax.axis_index("s")
        base_h = s * (2 * eo)
        base_o = base_h + c * eo

        cp_ia1 = pltpu.async_copy(a_hbm.at[pl.ds(base_h, eo)], ia1_v, sem_ia1)
        cp_ib1 = pltpu.async_copy(b_hbm.at[pl.ds(base_h, eo)], ib1_v, sem_ib1)
        cp_ones = pltpu.async_copy(ones_hbm, ones_v, sem_ones)
        cp_ia2 = pltpu.async_copy(
            a_hbm.at[pl.ds(base_h + eo, eo)], ia2_v, sem_ia2)
        cp_ib2 = pltpu.async_copy(
            b_hbm.at[pl.ds(base_h + eo, eo)], ib2_v, sem_ib2)

        @pl.when(s == 0)
        def _():
            pltpu.sync_copy(zeros_hbm, deg_a_sh)

        @pl.when(s == 1)
        def _():
            pltpu.sync_copy(zeros_hbm, deg_b_sh)

        plsc.subcore_barrier()

        cp_ia1.wait()
        cp_ones.wait()
        sc1 = pltpu.async_copy(ones_v, deg_a_sh.at[ia1_v], sem_sc, add=True)
        cp_ib1.wait()
        sc2 = pltpu.async_copy(ones_v, deg_b_sh.at[ib1_v], sem_sc, add=True)
        cp_ia2.wait()
        sc3 = pltpu.async_copy(ones_v, deg_a_sh.at[ia2_v], sem_sc, add=True)
        cp_ib2.wait()
        sc4 = pltpu.async_copy(ones_v, deg_b_sh.at[ib2_v], sem_sc, add=True)
        sc1.wait()
        sc2.wait()
        sc3.wait()
        sc4.wait()
        plsc.subcore_barrier()

        cp_da = pltpu.async_copy(deg_a_sh, da_v, sem_da)
        cp_db = pltpu.async_copy(deg_b_sh, db_v, sem_db)
        cp_da.wait()
        cp_db.wait()

        def emit(ea_v, eb_v):
            @plsc.parallel_loop(0, eo, step=L, unroll=8)
            def _(i):
                sl = pl.ds(i, L)
                da = plsc.load_gather(da_v, [ea_v[sl]])
                db = plsc.load_gather(db_v, [eb_v[sl]])
                out_v[sl] = _rsqrt_f32(da * db)

        @pl.when(c == 0)
        def _():
            emit(ia1_v, ib1_v)

        @pl.when(c == 1)
        def _():
            emit(ia2_v, ib2_v)

        pltpu.sync_copy(out_v, out_hbm.at[pl.ds(base_o, eo)])

    return norm_kernel(ref_a, ref_b, ones, zeros)


def kernel(X, ref_a, ref_b):
    n_nodes = X.shape[0]
    n_edges = ref_a.shape[0]
    ones = jnp.ones((n_edges // (NC * NS),), jnp.float32)
    zeros = jnp.zeros((n_nodes,), jnp.float32)
    return _norm_sc(
        ref_a.astype(jnp.int32),
        ref_b.astype(jnp.int32),
        ones,
        zeros,
        n_nodes=n_nodes,
        n_edges=n_edges,
    )


# speedup vs baseline: 113.2407x; 1.1306x over previous
import functools

import jax
import jax.numpy as jnp
from jax import lax
from jax.experimental import pallas as pl
from jax.experimental.pallas import tpu as pltpu
from jax.experimental.pallas import tpu_sc as plsc

NC = 2
NS = 16
L = 16


def _rsqrt_f32(x):
    xi = plsc.bitcast(x, jnp.int32)
    y = plsc.bitcast(jnp.int32(0x5F3759DF) - (xi >> 1), jnp.float32)
    for _ in range(2):
        y = y * (1.5 - 0.5 * x * y * y)
    return y


@functools.partial(jax.jit, static_argnames=("n_nodes", "n_edges"))
def _norm_sc(ref_a, ref_b, ones, zeros, *, n_nodes, n_edges):
    eo = n_edges // (NC * NS)

    mesh = plsc.VectorSubcoreMesh(core_axis_name="c", subcore_axis_name="s")

    @functools.partial(
        pl.kernel,
        out_type=(
            jax.ShapeDtypeStruct((n_edges,), jnp.float32),
            jax.ShapeDtypeStruct((NC, n_nodes), jnp.float32),
        ),
        mesh=mesh,
        compiler_params=pltpu.CompilerParams(needs_layout_passes=False),
        scratch_types=[
            pltpu.VMEM_SHARED((n_nodes,), jnp.float32),
            pltpu.VMEM((eo,), jnp.int32),
            pltpu.VMEM((eo,), jnp.int32),
            pltpu.VMEM((eo,), jnp.int32),
            pltpu.VMEM((eo,), jnp.float32),
            pltpu.VMEM((n_nodes,), jnp.float32),
            pltpu.VMEM((n_nodes,), jnp.float32),
            pltpu.VMEM((eo,), jnp.float32),
            pltpu.SemaphoreType.DMA,
            pltpu.SemaphoreType.DMA,
            pltpu.SemaphoreType.DMA,
            pltpu.SemaphoreType.DMA,
            pltpu.SemaphoreType.DMA,
            pltpu.SemaphoreType.DMA,
            pltpu.SemaphoreType.DMA,
            pltpu.SemaphoreType.REGULAR,
        ],
    )
    def norm_kernel(a_hbm, b_hbm, ones_hbm, zeros_hbm, out_hbm, pub_hbm,
                    deg_sh, i1_v, i2_v, io_v, ones_v, down_v, doth_v, out_v,
                    sem_i1, sem_i2, sem_io, sem_ones, sem_sc,
                    sem_down, sem_doth, xsem):
        c = lax.axis_index("c")
        s = lax.axis_index("s")
        base_h = s * (2 * eo)
        base_o = base_h + c * eo

        cp_ones = pltpu.async_copy(ones_hbm, ones_v, sem_ones)

        def hist(src_hbm, oth_hbm):
            cp_i1 = pltpu.async_copy(
                src_hbm.at[pl.ds(base_h, eo)], i1_v, sem_i1)
            cp_i2 = pltpu.async_copy(
                src_hbm.at[pl.ds(base_h + eo, eo)], i2_v, sem_i2)
            cp_io = pltpu.async_copy(
                oth_hbm.at[pl.ds(base_o, eo)], io_v, sem_io)

            @pl.when(s == 0)
            def _():
                pltpu.sync_copy(zeros_hbm, deg_sh)

            plsc.subcore_barrier()

            cp_i1.wait()
            cp_ones.wait()
            sc1 = pltpu.async_copy(ones_v, deg_sh.at[i1_v], sem_sc, add=True)
            cp_i2.wait()
            sc2 = pltpu.async_copy(ones_v, deg_sh.at[i2_v], sem_sc, add=True)
            sc1.wait()
            sc2.wait()
            plsc.subcore_barrier()

            @pl.when(s == 0)
            def _():
                pltpu.sync_copy(deg_sh, pub_hbm.at[c])
                pl.semaphore_signal(xsem, 1, core_index=1 - c)
                pl.semaphore_wait(xsem, 1)

            plsc.subcore_barrier()
            cp_io.wait()

        @pl.when(c == 0)
        def _():
            hist(a_hbm, b_hbm)

        @pl.when(c == 1)
        def _():
            hist(b_hbm, a_hbm)

        cp_down = pltpu.async_copy(deg_sh, down_v, sem_down)
        cp_doth = pltpu.async_copy(pub_hbm.at[1 - c], doth_v, sem_doth)
        cp_down.wait()
        cp_doth.wait()

        def emit(own_idx_v):
            @plsc.parallel_loop(0, eo, step=L, unroll=8)
            def _(i):
                sl = pl.ds(i, L)
                down = plsc.load_gather(down_v, [own_idx_v[sl]])
                doth = plsc.load_gather(doth_v, [io_v[sl]])
                out_v[sl] = _rsqrt_f32(down * doth)

        @pl.when(c == 0)
        def _():
            emit(i1_v)

        @pl.when(c == 1)
        def _():
            emit(i2_v)

        pltpu.sync_copy(out_v, out_hbm.at[pl.ds(base_o, eo)])

    return norm_kernel(ref_a, ref_b, ones, zeros)[0]


def kernel(X, ref_a, ref_b):
    n_nodes = X.shape[0]
    n_edges = ref_a.shape[0]
    ones = jnp.ones((n_edges // (NC * NS),), jnp.float32)
    zeros = jnp.zeros((n_nodes,), jnp.float32)
    return _norm_sc(
        ref_a.astype(jnp.int32),
        ref_b.astype(jnp.int32),
        ones,
        zeros,
        n_nodes=n_nodes,
        n_edges=n_edges,
    )
